# baseline (speedup 1.0000x reference)
import jax
import jax.numpy as jnp
from jax import lax
from jax.experimental import pallas as pl
from jax.experimental.pallas import tpu as pltpu
from jax.experimental.pallas import tpu_sc as plsc

B = 16
T = 64
V = 30000
A = 256
L = 16
ROWS = T // 2
RB = 32
NEG = -3.4e38


def _lane_permute(x, idx):
    dnums = lax.GatherDimensionNumbers(
        offset_dims=(), collapsed_slice_dims=(0,), start_index_map=(0,))
    return lax.gather(x, idx[:, None], dnums, slice_sizes=(1,),
                      mode=lax.GatherScatterMode.PROMISE_IN_BOUNDS)


def _copy_body(in_ref, out_ref):
    out_ref[...] = in_ref[...]
    head = in_ref[:, 0:128]
    lane = lax.broadcasted_iota(jnp.int32, (RB, 128), 1)
    out_ref[:, 0:128] = jnp.where(lane == 1, jnp.float32(0.0), head)


def _tc_copy(dec2):
    n = (B * T) // RB
    return pl.pallas_call(
        _copy_body,
        out_shape=jax.ShapeDtypeStruct((B * T, V), jnp.float32),
        grid=(n,),
        in_specs=[pl.BlockSpec((RB, V), lambda i: (i, 0))],
        out_specs=pl.BlockSpec((RB, V), lambda i: (i, 0)),
    )(dec2)


def _sc_patch_body(dec_hbm, attn_hbm, iseq_hbm, ct_hbm, ref_hbm,
                   iseq_v, ids_v, colmax_v, attn_v,
                   fidx_v, pval_v, dcell_v, gsem, ssem):
    core = lax.axis_index("c")
    sub = lax.axis_index("s")
    b = sub
    t0 = core * ROWS
    iota = lax.iota(jnp.int32, L)

    pltpu.sync_copy(iseq_hbm.at[b], iseq_v)
    for j in range(2):
        pltpu.async_copy(ct_hbm.at[iseq_v.at[j]],
                         ids_v.at[pl.ds(j * 128, 128)], gsem).wait()

    pltpu.sync_copy(attn_hbm.at[b], attn_v)
    for c in range(A // L):
        colmax_v[pl.ds(c * L, L)] = attn_v[0, pl.ds(c * L, L)]

    def _red_body(tt, _):
        for c in range(A // L):
            sl = pl.ds(c * L, L)
            colmax_v[sl] = jnp.maximum(colmax_v[sl], attn_v[tt, sl])
        return ()

    lax.fori_loop(1, T, _red_body, (), unroll=False)

    row_in_chunk = lax.shift_right_logical(iota, 2)
    k_of_lane = jnp.bitwise_and(iota, 3)
    for g in range(ROWS // 4):
        tloc = t0 + 4 * g + row_in_chunk
        aidx = tloc + k_of_lane * T
        idv = plsc.load_gather(ids_v, [aidx])
        valv = plsc.load_gather(colmax_v, [aidx])
        fidx_v[pl.ds(g * L, L)] = (b * T + tloc) * V + idv
        lane_base = iota - k_of_lane
        for r in (1, 2, 3):
            rot = lane_base + jnp.bitwise_and(iota + r, 3)
            idr = _lane_permute(idv, rot)
            vr = _lane_permute(valv, rot)
            valv = jnp.maximum(valv, jnp.where(idr == idv, vr, NEG))
        pval_v[pl.ds(g * L, L)] = valv

    pltpu.async_copy(dec_hbm.at[fidx_v], dcell_v, gsem).wait()
    for g in range(ROWS // 4):
        sl = pl.ds(g * L, L)
        tloc = t0 + 4 * g + row_in_chunk
        valv = jnp.maximum(pval_v[sl], dcell_v[sl])
        is_kill = fidx_v[sl] == (b * T + tloc) * V + 1
        pval_v[sl] = jnp.where(is_kill, jnp.float32(0.0), valv)

    pltpu.async_copy(pval_v, ref_hbm.at[fidx_v], ssem).wait()


def _make_sc_patch():
    mesh = plsc.VectorSubcoreMesh(core_axis_name="c", subcore_axis_name="s")
    return pl.kernel(
        _sc_patch_body,
        out_type=(),
        mesh=mesh,
        compiler_params=pltpu.CompilerParams(needs_layout_passes=False),
        scratch_types=[
            pltpu.VMEM((2, 128), jnp.int32),
            pltpu.VMEM((A,), jnp.int32),
            pltpu.VMEM((A,), jnp.float32),
            pltpu.VMEM((T, A), jnp.float32),
            pltpu.VMEM((2 * T,), jnp.int32),
            pltpu.VMEM((2 * T,), jnp.float32),
            pltpu.VMEM((2 * T,), jnp.float32),
            pltpu.SemaphoreType.DMA,
            pltpu.SemaphoreType.DMA,
        ],
    )


@jax.jit
def _pointer_generator(dec2, attn, iseq2, ct):
    out0 = _tc_copy(dec2)
    ref = jax.new_ref(out0.reshape(B * T * V))
    _make_sc_patch()(dec2.reshape(B * T * V), attn, iseq2, ct, ref)
    return jax.freeze(ref)


def kernel(decoder_outputs, attention_scores, input_sequence,
           repeat_idx, repeat_idx2, convert_table):
    del repeat_idx, repeat_idx2
    iseq2 = input_sequence.reshape(B, 2, 128)
    dec2 = decoder_outputs.reshape(B * T, V)
    out = _pointer_generator(dec2, attention_scores, iseq2, convert_table)
    return out.reshape(B, T, V)


# speedup vs baseline: 5.9465x; 5.9465x over previous
import jax
import jax.numpy as jnp
from jax import lax
from jax.experimental import pallas as pl
from jax.experimental.pallas import tpu as pltpu
from jax.experimental.pallas import tpu_sc as plsc

B = 16
T = 64
V = 30000
A = 256
L = 16
NB = 3
ROWS = T // 2


def _sc_body(dec_hbm, attn_hbm, iseq_hbm, ct_hbm, out_hbm,
             iseq_v, ids_v, colmax_v, attn_v, row0, row1, row2,
             gsem, asem, in_sems, out_sems):
    core = lax.axis_index("c")
    sub = lax.axis_index("s")
    b = sub
    t0 = core * ROWS
    iota = lax.iota(jnp.int32, L)
    rows = (row0, row1, row2)

    attn_cp = pltpu.async_copy(attn_hbm.at[b], attn_v, asem)
    in_descs = {}
    out_descs = {}
    for i in range(NB - 1):
        in_descs[i] = pltpu.async_copy(dec_hbm.at[b, t0 + i], rows[i],
                                       in_sems[i])

    pltpu.sync_copy(iseq_hbm.at[b], iseq_v)
    for j in range(2):
        pltpu.async_copy(ct_hbm.at[iseq_v.at[j]],
                         ids_v.at[pl.ds(j * 128, 128)], gsem).wait()

    attn_cp.wait()
    for c in range(A // L):
        colmax_v[pl.ds(c * L, L)] = attn_v[0, pl.ds(c * L, L)]

    def _red_body(tt, _):
        for c in range(A // L):
            sl = pl.ds(c * L, L)
            colmax_v[sl] = jnp.maximum(colmax_v[sl], attn_v[tt, sl])
        return ()

    lax.fori_loop(1, T, _red_body, (), unroll=False)

    for j in range(ROWS):
        buf = j % NB
        pj = j + NB - 1
        if pj < ROWS:
            pbuf = pj % NB
            if pj - NB >= 0:
                out_descs[pj - NB].wait()
            in_descs[pj] = pltpu.async_copy(dec_hbm.at[b, t0 + pj],
                                            rows[pbuf], in_sems[pbuf])
        in_descs[j].wait()
        row_v = rows[buf]
        t = t0 + j

        idx4 = t + jnp.minimum(iota, 3) * T
        ids4 = plsc.load_gather(ids_v, [idx4])
        vals4 = plsc.load_gather(colmax_v, [idx4])
        for k in range(A // T):
            old = plsc.load_gather(row_v, [ids4])
            new = jnp.maximum(old, vals4)
            plsc.store_scatter(row_v, [ids4], new, mask=(iota == k))

        head = row_v[pl.ds(0, L)]
        row_v[pl.ds(0, L)] = jnp.where(iota == 1, jnp.float32(0.0), head)

        out_descs[j] = pltpu.async_copy(row_v, out_hbm.at[b, t],
                                        out_sems[buf])
    for j in range(ROWS - NB, ROWS):
        out_descs[j].wait()


@jax.jit
def _pointer_generator_sc(dec, attn, iseq2, ct):
    mesh = plsc.VectorSubcoreMesh(core_axis_name="c", subcore_axis_name="s")
    return pl.kernel(
        _sc_body,
        out_type=jax.ShapeDtypeStruct((B, T, V), jnp.float32),
        mesh=mesh,
        compiler_params=pltpu.CompilerParams(needs_layout_passes=False),
        scratch_types=[
            pltpu.VMEM((2, 128), jnp.int32),
            pltpu.VMEM((A,), jnp.int32),
            pltpu.VMEM((A,), jnp.float32),
            pltpu.VMEM((T, A), jnp.float32),
            pltpu.VMEM((V,), jnp.float32),
            pltpu.VMEM((V,), jnp.float32),
            pltpu.VMEM((V,), jnp.float32),
            pltpu.SemaphoreType.DMA,
            pltpu.SemaphoreType.DMA,
            [pltpu.SemaphoreType.DMA] * NB,
            [pltpu.SemaphoreType.DMA] * NB,
        ],
    )(dec, attn, iseq2, ct)


def kernel(decoder_outputs, attention_scores, input_sequence,
           repeat_idx, repeat_idx2, convert_table):
    del repeat_idx, repeat_idx2
    iseq2 = input_sequence.reshape(B, 2, 128)
    return _pointer_generator_sc(decoder_outputs, attention_scores,
                                 iseq2, convert_table)
